# Initial kernel scaffold; baseline (speedup 1.0000x reference)
import jax
import jax.numpy as jnp
from jax.experimental import pallas as pl
from jax.experimental.pallas import tpu as pltpu

_COMMITMENT_COST = 0.25
_K = 8192
_D = 32
_N = 8192
_R = 128


def _vq_body(x_ref, xsq_ref, w_ref, wsq_ref,
             enc_ref, q_ref, loss_ref, perp_ref,
             counts_ref, elat_ref):
    i = pl.program_id(0)
    x = x_ref[...]
    w = w_ref[...]
    mm = jax.lax.dot_general(x, w, (((1,), (1,)), ((), ())),
                             preferred_element_type=jnp.float32)
    dist = (xsq_ref[...] + wsq_ref[...]) - 2.0 * mm
    minval = jnp.min(dist, axis=1, keepdims=True)
    col = jax.lax.broadcasted_iota(jnp.int32, dist.shape, 1)
    idx = jnp.min(jnp.where(dist == minval, col, _K), axis=1, keepdims=True)
    enc = jnp.where(col == idx, 1.0, 0.0).astype(jnp.float32)
    enc_ref[...] = enc
    q = jax.lax.dot_general(enc, w, (((1,), (0,)), ((), ())),
                            preferred_element_type=jnp.float32)
    q_ref[...] = x + (q - x)

    @pl.when(i == 0)
    def _init():
        counts_ref[...] = jnp.zeros_like(counts_ref)
        elat_ref[0, 0] = 0.0

    counts_ref[...] += jnp.sum(enc, axis=0, keepdims=True)
    d = q - x
    elat_ref[0, 0] += jnp.sum(d * d)

    @pl.when(i == pl.num_programs(0) - 1)
    def _finish():
        avg = counts_ref[...] * (1.0 / _N)
        perp_ref[0, 0] = jnp.exp(-jnp.sum(avg * jnp.log(avg + 1e-10)))
        loss_ref[0, 0] = (_COMMITMENT_COST / (_N * _D)) * elat_ref[0, 0]


def kernel(inputs, emb_w):
    flat = inputs.reshape(-1, _D)
    x_sq = jnp.sum(flat ** 2, axis=1, keepdims=True)
    w_sq = jnp.sum(emb_w ** 2, axis=1).reshape(1, _K)

    grid = _N // _R
    enc, q, loss, perp = pl.pallas_call(
        _vq_body,
        grid=(grid,),
        in_specs=[
            pl.BlockSpec((_R, _D), lambda i: (i, 0)),
            pl.BlockSpec((_R, 1), lambda i: (i, 0)),
            pl.BlockSpec((_K, _D), lambda i: (0, 0)),
            pl.BlockSpec((1, _K), lambda i: (0, 0)),
        ],
        out_specs=[
            pl.BlockSpec((_R, _K), lambda i: (i, 0)),
            pl.BlockSpec((_R, _D), lambda i: (i, 0)),
            pl.BlockSpec(memory_space=pltpu.SMEM),
            pl.BlockSpec(memory_space=pltpu.SMEM),
        ],
        out_shape=[
            jax.ShapeDtypeStruct((_N, _K), jnp.float32),
            jax.ShapeDtypeStruct((_N, _D), jnp.float32),
            jax.ShapeDtypeStruct((1, 1), jnp.float32),
            jax.ShapeDtypeStruct((1, 1), jnp.float32),
        ],
        scratch_shapes=[
            pltpu.VMEM((1, _K), jnp.float32),
            pltpu.SMEM((1, 1), jnp.float32),
        ],
        compiler_params=pltpu.CompilerParams(
            dimension_semantics=("arbitrary",),
        ),
    )(flat, x_sq, emb_w, w_sq)

    return (loss[0, 0], q.reshape(inputs.shape), perp[0, 0], enc)


# speedup vs baseline: 6.5393x; 6.5393x over previous
import jax
import jax.numpy as jnp
from jax.experimental import pallas as pl
from jax.experimental.pallas import tpu as pltpu

_COMMITMENT_COST = 0.25
_K = 8192
_D = 32
_N = 8192
_R = 128


def _vq_body(x_ref, idx_ref, w_ref,
             enc_ref, q_ref, loss_ref, perp_ref,
             counts_ref, elat_ref):
    i = pl.program_id(0)
    x = x_ref[...]
    w = w_ref[...]
    col = jax.lax.broadcasted_iota(jnp.int32, (_R, _K), 1)
    enc = jnp.where(col == idx_ref[...], 1.0, 0.0).astype(jnp.float32)
    enc_ref[...] = enc
    q = jax.lax.dot_general(enc, w, (((1,), (0,)), ((), ())),
                            preferred_element_type=jnp.float32)
    q_ref[...] = x + (q - x)

    @pl.when(i == 0)
    def _init():
        counts_ref[...] = jnp.zeros_like(counts_ref)
        elat_ref[0, 0] = 0.0

    counts_ref[...] += jnp.sum(enc, axis=0, keepdims=True)
    d = q - x
    elat_ref[0, 0] += jnp.sum(d * d)

    @pl.when(i == pl.num_programs(0) - 1)
    def _finish():
        avg = counts_ref[...] * (1.0 / _N)
        perp_ref[0, 0] = jnp.exp(-jnp.sum(avg * jnp.log(avg + 1e-10)))
        loss_ref[0, 0] = (_COMMITMENT_COST / (_N * _D)) * elat_ref[0, 0]


def kernel(inputs, emb_w):
    flat = inputs.reshape(-1, _D)
    input_sq_sum = jnp.sum(flat ** 2, axis=1, keepdims=True)
    emb_wt_sq_sum = jnp.sum(emb_w ** 2, axis=1)
    distances = input_sq_sum + emb_wt_sq_sum - 2.0 * jnp.matmul(flat, emb_w.T)
    idx = jnp.argmin(distances, axis=1).astype(jnp.int32).reshape(_N, 1)

    grid = _N // _R
    enc, q, loss, perp = pl.pallas_call(
        _vq_body,
        grid=(grid,),
        in_specs=[
            pl.BlockSpec((_R, _D), lambda i: (i, 0)),
            pl.BlockSpec((_R, 1), lambda i: (i, 0)),
            pl.BlockSpec((_K, _D), lambda i: (0, 0)),
        ],
        out_specs=[
            pl.BlockSpec((_R, _K), lambda i: (i, 0)),
            pl.BlockSpec((_R, _D), lambda i: (i, 0)),
            pl.BlockSpec(memory_space=pltpu.SMEM),
            pl.BlockSpec(memory_space=pltpu.SMEM),
        ],
        out_shape=[
            jax.ShapeDtypeStruct((_N, _K), jnp.float32),
            jax.ShapeDtypeStruct((_N, _D), jnp.float32),
            jax.ShapeDtypeStruct((1, 1), jnp.float32),
            jax.ShapeDtypeStruct((1, 1), jnp.float32),
        ],
        scratch_shapes=[
            pltpu.VMEM((1, _K), jnp.float32),
            pltpu.SMEM((1, 1), jnp.float32),
        ],
        compiler_params=pltpu.CompilerParams(
            dimension_semantics=("arbitrary",),
        ),
    )(flat, idx, emb_w)

    return (loss[0, 0], q.reshape(inputs.shape), perp[0, 0], enc)


# speedup vs baseline: 6.9499x; 1.0628x over previous
import jax
import jax.numpy as jnp
from jax.experimental import pallas as pl
from jax.experimental.pallas import tpu as pltpu

_COMMITMENT_COST = 0.25
_K = 8192
_D = 32
_N = 8192
_R = 256


def _vq_body(x_ref, idx_ref, w_ref,
             enc_ref, q_ref, loss_ref, perp_ref,
             counts_ref, elat_ref):
    i = pl.program_id(0)
    x = x_ref[...]
    w = w_ref[...]
    col = jax.lax.broadcasted_iota(jnp.int32, (_R, _K), 1)
    enc = jnp.where(col == idx_ref[...], 1.0, 0.0).astype(jnp.float32)
    enc_ref[...] = enc
    q = jax.lax.dot_general(enc, w, (((1,), (0,)), ((), ())),
                            preferred_element_type=jnp.float32)
    q_ref[...] = x + (q - x)

    @pl.when(i == 0)
    def _init():
        counts_ref[...] = jnp.zeros_like(counts_ref)
        elat_ref[0, 0] = 0.0

    counts_ref[...] += jnp.sum(enc, axis=0, keepdims=True)
    d = q - x
    elat_ref[0, 0] += jnp.sum(d * d)

    @pl.when(i == pl.num_programs(0) - 1)
    def _finish():
        avg = counts_ref[...] * (1.0 / _N)
        perp_ref[0, 0] = jnp.exp(-jnp.sum(avg * jnp.log(avg + 1e-10)))
        loss_ref[0, 0] = (_COMMITMENT_COST / (_N * _D)) * elat_ref[0, 0]


def kernel(inputs, emb_w):
    flat = inputs.reshape(-1, _D)
    input_sq_sum = jnp.sum(flat ** 2, axis=1, keepdims=True)
    emb_wt_sq_sum = jnp.sum(emb_w ** 2, axis=1)
    distances = input_sq_sum + emb_wt_sq_sum - 2.0 * jnp.matmul(flat, emb_w.T)
    idx = jnp.argmin(distances, axis=1).astype(jnp.int32).reshape(_N, 1)

    grid = _N // _R
    enc, q, loss, perp = pl.pallas_call(
        _vq_body,
        grid=(grid,),
        in_specs=[
            pl.BlockSpec((_R, _D), lambda i: (i, 0)),
            pl.BlockSpec((_R, 1), lambda i: (i, 0)),
            pl.BlockSpec((_K, _D), lambda i: (0, 0)),
        ],
        out_specs=[
            pl.BlockSpec((_R, _K), lambda i: (i, 0)),
            pl.BlockSpec((_R, _D), lambda i: (i, 0)),
            pl.BlockSpec(memory_space=pltpu.SMEM),
            pl.BlockSpec(memory_space=pltpu.SMEM),
        ],
        out_shape=[
            jax.ShapeDtypeStruct((_N, _K), jnp.float32),
            jax.ShapeDtypeStruct((_N, _D), jnp.float32),
            jax.ShapeDtypeStruct((1, 1), jnp.float32),
            jax.ShapeDtypeStruct((1, 1), jnp.float32),
        ],
        scratch_shapes=[
            pltpu.VMEM((1, _K), jnp.float32),
            pltpu.SMEM((1, 1), jnp.float32),
        ],
        compiler_params=pltpu.CompilerParams(
            dimension_semantics=("arbitrary",),
        ),
    )(flat, idx, emb_w)

    return (loss[0, 0], q.reshape(inputs.shape), perp[0, 0], enc)


# speedup vs baseline: 7.0660x; 1.0167x over previous
import jax
import jax.numpy as jnp
from jax.experimental import pallas as pl
from jax.experimental.pallas import tpu as pltpu

_COMMITMENT_COST = 0.25
_K = 8192
_D = 32
_N = 8192
_R = 512


def _vq_body(x_ref, idx_ref, w_ref,
             enc_ref, q_ref, loss_ref, perp_ref,
             counts_ref, elat_ref):
    i = pl.program_id(0)
    x = x_ref[...]
    w = w_ref[...]
    col = jax.lax.broadcasted_iota(jnp.int32, (_R, _K), 1)
    enc = jnp.where(col == idx_ref[...], 1.0, 0.0).astype(jnp.float32)
    enc_ref[...] = enc
    q = jax.lax.dot_general(enc, w, (((1,), (0,)), ((), ())),
                            preferred_element_type=jnp.float32)
    q_ref[...] = x + (q - x)

    @pl.when(i == 0)
    def _init():
        counts_ref[...] = jnp.zeros_like(counts_ref)
        elat_ref[0, 0] = 0.0

    counts_ref[...] += jnp.sum(enc, axis=0, keepdims=True)
    d = q - x
    elat_ref[0, 0] += jnp.sum(d * d)

    @pl.when(i == pl.num_programs(0) - 1)
    def _finish():
        avg = counts_ref[...] * (1.0 / _N)
        perp_ref[0, 0] = jnp.exp(-jnp.sum(avg * jnp.log(avg + 1e-10)))
        loss_ref[0, 0] = (_COMMITMENT_COST / (_N * _D)) * elat_ref[0, 0]


def kernel(inputs, emb_w):
    flat = inputs.reshape(-1, _D)
    input_sq_sum = jnp.sum(flat ** 2, axis=1, keepdims=True)
    emb_wt_sq_sum = jnp.sum(emb_w ** 2, axis=1)
    distances = input_sq_sum + emb_wt_sq_sum - 2.0 * jnp.matmul(flat, emb_w.T)
    idx = jnp.argmin(distances, axis=1).astype(jnp.int32).reshape(_N, 1)

    grid = _N // _R
    enc, q, loss, perp = pl.pallas_call(
        _vq_body,
        grid=(grid,),
        in_specs=[
            pl.BlockSpec((_R, _D), lambda i: (i, 0)),
            pl.BlockSpec((_R, 1), lambda i: (i, 0)),
            pl.BlockSpec((_K, _D), lambda i: (0, 0)),
        ],
        out_specs=[
            pl.BlockSpec((_R, _K), lambda i: (i, 0)),
            pl.BlockSpec((_R, _D), lambda i: (i, 0)),
            pl.BlockSpec(memory_space=pltpu.SMEM),
            pl.BlockSpec(memory_space=pltpu.SMEM),
        ],
        out_shape=[
            jax.ShapeDtypeStruct((_N, _K), jnp.float32),
            jax.ShapeDtypeStruct((_N, _D), jnp.float32),
            jax.ShapeDtypeStruct((1, 1), jnp.float32),
            jax.ShapeDtypeStruct((1, 1), jnp.float32),
        ],
        scratch_shapes=[
            pltpu.VMEM((1, _K), jnp.float32),
            pltpu.SMEM((1, 1), jnp.float32),
        ],
        compiler_params=pltpu.CompilerParams(
            dimension_semantics=("arbitrary",),
        ),
    )(flat, idx, emb_w)

    return (loss[0, 0], q.reshape(inputs.shape), perp[0, 0], enc)
